# Initial kernel scaffold; baseline (speedup 1.0000x reference)
import jax
import jax.numpy as jnp
from jax.experimental import pallas as pl


def kernel(x, shapes, scales):
    raise NotImplementedError("write your pallas kernel here")


# speedup vs baseline: 1084.4483x; 1084.4483x over previous
import functools

import jax
import jax.numpy as jnp
from jax import lax
from jax.experimental import pallas as pl
from jax.experimental.pallas import tpu as pltpu
from jax.experimental.pallas import tpu_sc as plsc

L = 8
DIM = 16
N_BETAS = 2**L - 1
N_BOUND = 2 ** (L - 1) - 1
N_LEAF = 2 ** (L - 1)
N_POINTS = 131072

NC = 2
NS = 16
LANES = 16
NW = NC * NS
CHUNK = N_POINTS // NW
GROUPS = CHUNK // LANES


def _build_tables(shapes, scales):
    a = jnp.log1p(jnp.exp(shapes))
    b = jnp.log1p(jnp.exp(scales))
    samples = jax.random.beta(jax.random.key(42), a, b)
    lowers = [jnp.zeros(DIM, dtype=samples.dtype)]
    uppers = [jnp.ones(DIM, dtype=samples.dtype)]
    for j in range(1, N_BETAS):
        p = (j - 1) // 2
        split = lowers[p] + samples[:, p] * (uppers[p] - lowers[p])
        if j % 2 == 1:
            lowers.append(lowers[p])
            uppers.append(split)
        else:
            lowers.append(split + 1e-07)
            uppers.append(uppers[p])
    boundaries = jnp.stack([uppers[2 * bb + 1] for bb in range(N_BOUND)], axis=0)
    prods = [None] * N_BETAS
    prods[0] = samples[:, 0]
    for j in range(1, N_BETAS):
        prods[j] = prods[(j - 1) // 2] * samples[:, j]
    y_leaf = jnp.stack(prods[N_BOUND:N_BETAS], axis=0)
    b_leaf = jnp.stack(
        [uppers[j] - lowers[j] for j in range(N_BOUND, N_BETAS)], axis=0)
    log_b = jnp.log(jnp.maximum(b_leaf, 1e-05))
    t_leaf = jnp.log(jnp.maximum(y_leaf, 1e-05) - log_b)
    bnd_pad = jnp.concatenate(
        [jnp.zeros((1, DIM), boundaries.dtype), boundaries], axis=0)
    return bnd_pad, t_leaf


def _sc_body(x_hbm, bnd_hbm, t_hbm, out_hbm, x_v, bnd_v, t_v, out_v):
    c = lax.axis_index("c")
    s = lax.axis_index("s")
    wid = s * NC + c
    base = wid * CHUNK
    pltpu.sync_copy(bnd_hbm, bnd_v)
    pltpu.sync_copy(t_hbm, t_v)
    pltpu.sync_copy(x_hbm.at[pl.ds(base * DIM, CHUNK * DIM)], x_v)
    lane = lax.broadcasted_iota(jnp.int32, (LANES,), 0)

    def group(g, carry):
        pbase = (lane + g * LANES) * DIM
        acc = jnp.zeros((LANES,), jnp.float32)
        for d in range(DIM):
            xv = plsc.load_gather(x_v, [pbase + d])
            node = jnp.full((LANES,), DIM + d, jnp.int32)
            for _ in range(L - 1):
                bnd = plsc.load_gather(bnd_v, [node])
                node = 2 * node + jnp.where(xv <= bnd, -d, DIM - d)
            acc = acc + plsc.load_gather(t_v, [node - (N_BOUND + 1) * DIM])
        out_v[pl.ds(g * LANES, LANES)] = acc * (1.0 / DIM)
        return carry

    lax.fori_loop(0, GROUPS, group, 0)
    pltpu.sync_copy(out_v, out_hbm.at[pl.ds(base, CHUNK)])


@functools.cache
def _polya_sc():
    return functools.partial(
        pl.kernel,
        out_type=jax.ShapeDtypeStruct((N_POINTS,), jnp.float32),
        mesh=plsc.VectorSubcoreMesh(
            core_axis_name="c", subcore_axis_name="s",
            num_cores=NC, num_subcores=NS),
        compiler_params=pltpu.CompilerParams(needs_layout_passes=False),
        scratch_types=[
            pltpu.VMEM((CHUNK * DIM,), jnp.float32),
            pltpu.VMEM((N_LEAF * DIM,), jnp.float32),
            pltpu.VMEM((N_LEAF * DIM,), jnp.float32),
            pltpu.VMEM((CHUNK,), jnp.float32),
        ],
    )(_sc_body)


def kernel(x, shapes, scales):
    bnd, t_leaf = _build_tables(shapes, scales)
    return _polya_sc()(x.reshape(-1), bnd.reshape(-1), t_leaf.reshape(-1))


# speedup vs baseline: 1192.8819x; 1.1000x over previous
import functools

import jax
import jax.numpy as jnp
from jax import lax
from jax.experimental import pallas as pl
from jax.experimental.pallas import tpu as pltpu
from jax.experimental.pallas import tpu_sc as plsc

L = 8
DIM = 16
N_BETAS = 2**L - 1
N_BOUND = 2 ** (L - 1) - 1
N_LEAF = 2 ** (L - 1)
N_POINTS = 131072

NC = 2
NS = 16
LANES = 16
NW = NC * NS
CHUNK = N_POINTS // NW
GROUPS = CHUNK // LANES


def _build_tables(shapes, scales):
    a = jnp.log1p(jnp.exp(shapes))
    b = jnp.log1p(jnp.exp(scales))
    samples = jax.random.beta(jax.random.key(42), a, b)

    def interleave(u, v):
        return jnp.stack([u, v], axis=2).reshape(DIM, -1)

    low = jnp.zeros((DIM, 1), samples.dtype)
    up = jnp.ones((DIM, 1), samples.dtype)
    prod = samples[:, 0:1]
    bnds = []
    for lvl in range(L - 1):
        s_lvl = samples[:, 2**lvl - 1: 2 ** (lvl + 1) - 1]
        split = low + s_lvl * (up - low)
        bnds.append(split)
        low = interleave(low, split + 1e-07)
        up = interleave(split, up)
        child_s = samples[:, 2 ** (lvl + 1) - 1: 2 ** (lvl + 2) - 1]
        prod = jnp.repeat(prod, 2, axis=1) * child_s
    boundaries = jnp.concatenate(bnds, axis=1).T
    y_leaf = prod.T
    b_leaf = (up - low).T
    log_b = jnp.log(jnp.maximum(b_leaf, 1e-05))
    t_leaf = jnp.log(jnp.maximum(y_leaf, 1e-05) - log_b)
    bnd_pad = jnp.concatenate(
        [jnp.zeros((1, DIM), boundaries.dtype), boundaries], axis=0)
    return bnd_pad, t_leaf


def _sc_body(x_hbm, bnd_hbm, t_hbm, out_hbm, x_v, bnd_v, t_v, out_v):
    c = lax.axis_index("c")
    s = lax.axis_index("s")
    wid = s * NC + c
    base = wid * CHUNK
    pltpu.sync_copy(bnd_hbm, bnd_v)
    pltpu.sync_copy(t_hbm, t_v)
    pltpu.sync_copy(x_hbm.at[pl.ds(base * DIM, CHUNK * DIM)], x_v)
    lane = lax.broadcasted_iota(jnp.int32, (LANES,), 0)

    def group(g, carry):
        pbase = (lane + g * LANES) * DIM
        acc = jnp.zeros((LANES,), jnp.float32)
        for d in range(DIM):
            xv = plsc.load_gather(x_v, [pbase + d])
            node = jnp.full((LANES,), DIM + d, jnp.int32)
            for _ in range(L - 1):
                bnd = plsc.load_gather(bnd_v, [node])
                node = 2 * node + jnp.where(xv <= bnd, -d, DIM - d)
            acc = acc + plsc.load_gather(t_v, [node - (N_BOUND + 1) * DIM])
        out_v[pl.ds(g * LANES, LANES)] = acc * (1.0 / DIM)
        return carry

    lax.fori_loop(0, GROUPS, group, 0)
    pltpu.sync_copy(out_v, out_hbm.at[pl.ds(base, CHUNK)])


@functools.cache
def _polya_sc():
    return functools.partial(
        pl.kernel,
        out_type=jax.ShapeDtypeStruct((N_POINTS,), jnp.float32),
        mesh=plsc.VectorSubcoreMesh(
            core_axis_name="c", subcore_axis_name="s",
            num_cores=NC, num_subcores=NS),
        compiler_params=pltpu.CompilerParams(needs_layout_passes=False),
        scratch_types=[
            pltpu.VMEM((CHUNK * DIM,), jnp.float32),
            pltpu.VMEM((N_LEAF * DIM,), jnp.float32),
            pltpu.VMEM((N_LEAF * DIM,), jnp.float32),
            pltpu.VMEM((CHUNK,), jnp.float32),
        ],
    )(_sc_body)


def kernel(x, shapes, scales):
    bnd, t_leaf = _build_tables(shapes, scales)
    return _polya_sc()(x.reshape(-1), bnd.reshape(-1), t_leaf.reshape(-1))


# speedup vs baseline: 1458.9041x; 1.2230x over previous
import functools

import jax
import jax.numpy as jnp
from jax import lax
from jax.experimental import pallas as pl
from jax.experimental.pallas import tpu as pltpu
from jax.experimental.pallas import tpu_sc as plsc

L = 8
DIM = 16
N_BETAS = 2**L - 1
N_BOUND = 2 ** (L - 1) - 1
N_LEAF = 2 ** (L - 1)
N_POINTS = 131072
TSTRIDE = 17

NC = 2
NS = 16
LANES = 16
NW = NC * NS
CHUNK = N_POINTS // NW
GROUPS = CHUNK // LANES


def _build_tables(shapes, scales):
    a = jnp.log1p(jnp.exp(shapes))
    b = jnp.log1p(jnp.exp(scales))
    samples = jax.random.beta(jax.random.key(42), a, b)

    def interleave(u, v):
        return jnp.stack([u, v], axis=2).reshape(DIM, -1)

    low = jnp.zeros((DIM, 1), samples.dtype)
    up = jnp.ones((DIM, 1), samples.dtype)
    prod = samples[:, 0:1]
    bnds = []
    for lvl in range(L - 1):
        s_lvl = samples[:, 2**lvl - 1: 2 ** (lvl + 1) - 1]
        split = low + s_lvl * (up - low)
        bnds.append(split)
        low = interleave(low, split + 1e-07)
        up = interleave(split, up)
        child_s = samples[:, 2 ** (lvl + 1) - 1: 2 ** (lvl + 2) - 1]
        prod = jnp.repeat(prod, 2, axis=1) * child_s
    boundaries = jnp.concatenate(bnds, axis=1).T
    y_leaf = prod.T
    b_leaf = (up - low).T
    log_b = jnp.log(jnp.maximum(b_leaf, 1e-05))
    t_leaf = jnp.log(jnp.maximum(y_leaf, 1e-05) - log_b)
    bnd_pad = jnp.concatenate(
        [jnp.zeros((1, DIM), boundaries.dtype), boundaries], axis=0)
    bnd_pad = jnp.pad(bnd_pad, ((0, 0), (0, TSTRIDE - DIM)))
    t_pad = jnp.pad(t_leaf, ((0, 0), (0, TSTRIDE - DIM)))
    return bnd_pad, t_pad


def _sc_body(x_hbm, bnd_hbm, t_hbm, out_hbm, x_v, bnd_v, t_v, out_v):
    c = lax.axis_index("c")
    s = lax.axis_index("s")
    wid = s * NC + c
    base = wid * CHUNK
    pltpu.sync_copy(bnd_hbm, bnd_v)
    pltpu.sync_copy(t_hbm, t_v)
    pltpu.sync_copy(x_hbm.at[pl.ds(base * DIM, CHUNK * DIM)], x_v)
    lane = lax.broadcasted_iota(jnp.int32, (LANES,), 0)

    def group(g, carry):
        pbase = (lane + g * LANES) * DIM
        acc = jnp.zeros((LANES,), jnp.float32)
        for d in range(DIM):
            xv = plsc.load_gather(x_v, [pbase + d])
            node = jnp.full((LANES,), TSTRIDE + d, jnp.int32)
            for _ in range(L - 1):
                bnd = plsc.load_gather(bnd_v, [node])
                node = 2 * node + jnp.where(xv <= bnd, -d, TSTRIDE - d)
            acc = acc + plsc.load_gather(t_v, [node - (N_BOUND + 1) * TSTRIDE])
        out_v[pl.ds(g * LANES, LANES)] = acc * (1.0 / DIM)
        return carry

    lax.fori_loop(0, GROUPS, group, 0)
    pltpu.sync_copy(out_v, out_hbm.at[pl.ds(base, CHUNK)])


@functools.cache
def _polya_sc():
    return functools.partial(
        pl.kernel,
        out_type=jax.ShapeDtypeStruct((N_POINTS,), jnp.float32),
        mesh=plsc.VectorSubcoreMesh(
            core_axis_name="c", subcore_axis_name="s",
            num_cores=NC, num_subcores=NS),
        compiler_params=pltpu.CompilerParams(needs_layout_passes=False),
        scratch_types=[
            pltpu.VMEM((CHUNK * DIM,), jnp.float32),
            pltpu.VMEM((N_LEAF * TSTRIDE,), jnp.float32),
            pltpu.VMEM((N_LEAF * TSTRIDE,), jnp.float32),
            pltpu.VMEM((CHUNK,), jnp.float32),
        ],
    )(_sc_body)


def kernel(x, shapes, scales):
    bnd, t_leaf = _build_tables(shapes, scales)
    return _polya_sc()(x.reshape(-1), bnd.reshape(-1), t_leaf.reshape(-1))
